# baseline (speedup 1.0000x reference)
import functools

import jax
import jax.numpy as jnp
from jax import lax
from jax.experimental import pallas as pl
from jax.experimental.pallas import tpu as pltpu
from jax.experimental.pallas import tpu_sc as plsc

N = 10000
M = 5000
D = 128

NC = 2
NS = 16
NW = NC * NS
CH = 128


def _round_up(x, m):
    return (x + m - 1) // m * m


@functools.lru_cache(maxsize=None)
def _make_seg_pass(n_edges_pad, width, out_rows_pad, scaled, ch, grp):
    cpw = n_edges_pad // (NW * ch)
    ngroups = cpw // grp
    rps = out_rows_pad // NS
    mesh = plsc.VectorSubcoreMesh(
        core_axis_name="c", subcore_axis_name="s", num_cores=NC, num_subcores=NS
    )

    scratch = [
        pltpu.VMEM((grp, ch), jnp.int32),
        pltpu.VMEM((grp, ch), jnp.int32),
        pltpu.VMEM((ch, width), jnp.float32),
        pltpu.VMEM((ch, width), jnp.float32),
        pltpu.VMEM_SHARED((out_rows_pad, width), jnp.float32),
        pltpu.SemaphoreType.DMA,
        pltpu.SemaphoreType.DMA,
    ]
    if scaled:
        scratch.insert(2, pltpu.VMEM((grp, ch), jnp.float32))

    def body(table, gidx, sidx, vals, zinit, out, gi_g, si_g, *rest):
        if scaled:
            val_g, rows0, rows1, acc, sem0, sem1 = rest
        else:
            (rows0, rows1, acc, sem0, sem1) = rest
            val_g = None
        c = lax.axis_index("c")
        s = lax.axis_index("s")
        wid = c * NS + s
        c0 = pl.multiple_of(wid * cpw, grp)

        r0 = s * rps
        pltpu.sync_copy(zinit.at[pl.ds(r0, rps)], acc.at[pl.ds(r0, rps)])
        plsc.subcore_barrier()

        def scale_rows(rows_v, g):
            if not scaled:
                return

            def scale_blk(b, _):
                base16 = b * 16
                vv = val_g[g, pl.ds(base16, 16)]
                for i in range(16):
                    v = vv[i]
                    for j in range(width // 16):
                        sl = pl.ds(j * 16, 16)
                        rows_v[base16 + i, sl] = rows_v[base16 + i, sl] * v
                return 0

            lax.fori_loop(0, ch // 16, scale_blk, 0)

        def group(gi, _):
            gbase = pl.multiple_of(c0 + gi * grp, grp)
            pltpu.sync_copy(gidx.at[pl.ds(gbase, grp)], gi_g)
            pltpu.sync_copy(sidx.at[pl.ds(gbase, grp)], si_g)
            if scaled:
                pltpu.sync_copy(vals.at[pl.ds(gbase, grp)], val_g)

            pltpu.async_copy(table.at[gi_g.at[0]], rows0, sem0)
            bufs = (rows0, rows1)
            sems = (sem0, sem1)
            for g in range(grp):
                rb, sb = bufs[g % 2], sems[g % 2]
                if g + 1 < grp:
                    pltpu.async_copy(
                        table.at[gi_g.at[g + 1]], bufs[(g + 1) % 2], sems[(g + 1) % 2]
                    )
                pltpu.make_async_copy(table.at[gi_g.at[g]], rb, sb).wait()
                scale_rows(rb, g)
                pltpu.sync_copy(rb, acc.at[si_g.at[g]], add=True)
            return 0

        lax.fori_loop(0, ngroups, group, 0)
        plsc.subcore_barrier()
        pltpu.sync_copy(acc.at[pl.ds(r0, rps)], out.at[c, pl.ds(r0, rps)])

    if not scaled:
        def body_nv(table, gidx, sidx, zinit, out, *scr):
            return body(table, gidx, sidx, None, zinit, out, *scr)
        fn = body_nv
    else:
        fn = body

    return pl.kernel(
        fn,
        out_type=jax.ShapeDtypeStruct((NC, out_rows_pad, width), jnp.float32),
        mesh=mesh,
        scratch_types=scratch,
        compiler_params=pltpu.CompilerParams(use_tc_tiling_on_sc=False),
    )


def _seg_pass(table, gather_idx, scatter_idx, out_rows, vals=None, ch=CH, grp=8):
    n_e = gather_idx.shape[0]
    width = table.shape[1]
    n_pad = _round_up(n_e, NW * ch * grp)
    out_rows_pad = _round_up(out_rows + 1, NS * 8)
    junk = out_rows
    pad = n_pad - n_e
    gi = jnp.concatenate(
        [gather_idx.astype(jnp.int32), jnp.zeros((pad,), jnp.int32)]
    ).reshape(n_pad // ch, ch)
    si = jnp.concatenate(
        [scatter_idx.astype(jnp.int32), jnp.full((pad,), junk, jnp.int32)]
    ).reshape(n_pad // ch, ch)
    zinit = jnp.zeros((out_rows_pad, width), jnp.float32)
    k = _make_seg_pass(n_pad, width, out_rows_pad, vals is not None, ch, grp)
    if vals is not None:
        va = jnp.concatenate(
            [vals.astype(jnp.float32), jnp.zeros((pad,), jnp.float32)]
        ).reshape(n_pad // ch, ch)
        parts = k(table, gi, si, va, zinit)
    else:
        parts = k(table, gi, si, zinit)
    return parts[0] + parts[1]


_BM = 1000


def _dot(a, b):
    return jnp.dot(a, b, preferred_element_type=jnp.float32)


def _row_spec(bm, w):
    return pl.BlockSpec((bm, w), lambda i: (i, 0))


def _const_spec(r, c):
    return pl.BlockSpec((r, c), lambda i: (0, 0))


def _tc_psi1(S_v, sum_w, cnt, efeat, w1at, w1bt, b1):
    def body(sv, sw, ct, ef, wa, wb, b, out):
        tmp = _dot(ef[...], wb[...]) + b[...]
        rc = 1.0 / jnp.maximum(ct[...], 1.0)
        out[...] = (_dot(sv[...], wa[...]) + sw[...] * tmp) * rc

    m = S_v.shape[0]
    return pl.pallas_call(
        body,
        grid=(m // _BM,),
        in_specs=[
            _row_spec(_BM, D), _row_spec(_BM, 1), _row_spec(_BM, 1),
            _row_spec(_BM, D), _const_spec(D, D), _const_spec(D, D),
            _const_spec(1, D),
        ],
        out_specs=_row_spec(_BM, D),
        out_shape=jax.ShapeDtypeStruct((m, D), jnp.float32),
    )(S_v, sum_w, cnt, efeat, w1at, w1bt, b1)


def _tc_vout(vf_pre, cnt_src, wvt):
    def body(vp, ct, wv, out):
        rc = 1.0 / jnp.maximum(ct[...], 1.0)
        out[...] = jnp.maximum(_dot(vp[...] * rc, wv[...]), 0.0)

    n = vf_pre.shape[0]
    return pl.pallas_call(
        body,
        grid=(n // _BM,),
        in_specs=[_row_spec(_BM, D), _row_spec(_BM, 1), _const_spec(D, D)],
        out_specs=_row_spec(_BM, D),
        out_shape=jax.ShapeDtypeStruct((n, D), jnp.float32),
    )(vf_pre, cnt_src, wvt)


def _tc_eout(S_vo, ef3_pre, cnt, efeat, w2at, w2bt, b2, wet):
    def body(svo, e3p, ct, ef, wa, wb, b, we, out):
        ctv = ct[...]
        rc = 1.0 / jnp.maximum(ctv, 1.0)
        bb = (_dot(svo[...], wa[...]) + ctv * (_dot(ef[...], wb[...]) + b[...])) * rc
        e3 = e3p[...] * rc + bb
        out[...] = jnp.maximum(_dot(e3, we[...]), 0.0)

    m = S_vo.shape[0]
    return pl.pallas_call(
        body,
        grid=(m // _BM,),
        in_specs=[
            _row_spec(_BM, D), _row_spec(_BM, D), _row_spec(_BM, 1),
            _row_spec(_BM, D), _const_spec(D, D), _const_spec(D, D),
            _const_spec(1, D), _const_spec(D, D),
        ],
        out_specs=_row_spec(_BM, D),
        out_shape=jax.ShapeDtypeStruct((m, D), jnp.float32),
    )(S_vo, ef3_pre, cnt, efeat, w2at, w2bt, b2, wet)


def kernel(vfeat, efeat, invDV, invDE, in_src, in_dst, eMat_row, eMat_col,
           eMat_val, vMat_row, vMat_col, vMat_val, Wv, We, psi1_w, psi1_b,
           psi2_w, psi2_b):
    vfeat = vfeat.astype(jnp.float32)
    efeat = efeat.astype(jnp.float32)
    w1at = psi1_w[:, :D].T
    w1bt = psi1_w[:, D:].T
    w2at = psi2_w[:, :D].T
    w2bt = psi2_w[:, D:].T
    b1 = psi1_b.reshape(1, D)
    b2 = psi2_b.reshape(1, D)

    aug1 = jnp.concatenate(
        [vfeat * invDV[:, None], invDV[:, None], jnp.ones((N, 1), jnp.float32),
         jnp.zeros((N, 14), jnp.float32)], axis=1)
    p1 = _seg_pass(aug1, in_src, in_dst, M)
    S_v, sum_w, cnt = p1[:M, :D], p1[:M, D:D + 1], p1[:M, D + 1:D + 2]
    A = _tc_psi1(S_v, sum_w, cnt, efeat, w1at, w1bt, b1)

    As = _seg_pass(A, eMat_col, eMat_row, M, vals=eMat_val, grp=4)
    ef = As[:M, :D] + efeat

    aug3 = jnp.concatenate(
        [efeat * invDE[:, None], jnp.ones((M, 1), jnp.float32),
         jnp.zeros((M, 15), jnp.float32)], axis=1)
    p3 = _seg_pass(aug3, in_dst, in_src, N)
    vf2_pre, cnt_src = p3[:N, :D], p3[:N, D:D + 1]
    rcs = 1.0 / jnp.maximum(cnt_src, 1.0)

    vf_pre = _seg_pass(ef, in_dst, in_src, N, grp=16)
    vfo = _tc_vout(vf_pre[:N, :D], cnt_src, Wv.T)

    vf2 = _seg_pass(vf2_pre * rcs, vMat_col, vMat_row, N, vals=vMat_val, grp=4)

    tab45 = jnp.concatenate([vfo, vf2[:N, :D]], axis=1)
    p45 = _seg_pass(tab45, in_src, in_dst, M, ch=64, grp=8)
    S_vo, ef3_pre = p45[:M, :D], p45[:M, D:]
    efo = _tc_eout(S_vo, ef3_pre, cnt, efeat, w2at, w2bt, b2, We.T)

    return (vfo, efo)


# speedup vs baseline: 1.0887x; 1.0887x over previous
import functools

import jax
import jax.numpy as jnp
from jax import lax
from jax.experimental import pallas as pl
from jax.experimental.pallas import tpu as pltpu
from jax.experimental.pallas import tpu_sc as plsc

N = 10000
M = 5000
D = 128

NC = 2
NS = 16
NW = NC * NS
CH = 128


def _round_up(x, m):
    return (x + m - 1) // m * m


@functools.lru_cache(maxsize=None)
def _make_seg_pass(n_edges_pad, width, out_rows_pad, scaled, ch, grp):
    cpw = n_edges_pad // (NW * ch)
    ngroups = cpw // grp
    rps = out_rows_pad // NS
    mesh = plsc.VectorSubcoreMesh(
        core_axis_name="c", subcore_axis_name="s", num_cores=NC, num_subcores=NS
    )

    scratch = [
        pltpu.VMEM((grp, ch), jnp.int32),
        pltpu.VMEM((grp, ch), jnp.int32),
        pltpu.VMEM((ch, width), jnp.float32),
        pltpu.VMEM_SHARED((out_rows_pad, width), jnp.float32),
        pltpu.SemaphoreType.DMA,
    ]
    if scaled:
        scratch.insert(2, pltpu.VMEM((grp, ch), jnp.float32))

    def body(table, gidx, sidx, vals, zinit, out, gi_g, si_g, *rest):
        if scaled:
            val_g, rows0, acc, sem0 = rest
        else:
            (rows0, acc, sem0) = rest
            val_g = None
        c = lax.axis_index("c")
        s = lax.axis_index("s")
        wid = c * NS + s
        c0 = pl.multiple_of(wid * cpw, grp)

        r0 = s * rps
        pltpu.sync_copy(zinit.at[pl.ds(r0, rps)], acc.at[pl.ds(r0, rps)])
        plsc.subcore_barrier()

        def scale_rows(rows_v, g):
            if not scaled:
                return

            def scale_blk(b, _):
                base16 = b * 16
                vv = val_g[g, pl.ds(base16, 16)]
                for i in range(16):
                    v = vv[i]
                    for j in range(width // 16):
                        sl = pl.ds(j * 16, 16)
                        rows_v[base16 + i, sl] = rows_v[base16 + i, sl] * v
                return 0

            lax.fori_loop(0, ch // 16, scale_blk, 0)

        def group(gi, _):
            gbase = pl.multiple_of(c0 + gi * grp, grp)
            pltpu.sync_copy(gidx.at[pl.ds(gbase, grp)], gi_g)
            pltpu.sync_copy(sidx.at[pl.ds(gbase, grp)], si_g)
            if scaled:
                pltpu.sync_copy(vals.at[pl.ds(gbase, grp)], val_g)

            def chunk(g, _):
                pltpu.async_copy(table.at[gi_g.at[g]], rows0, sem0).wait()
                scale_rows(rows0, g)
                pltpu.sync_copy(rows0, acc.at[si_g.at[g]], add=True)
                return 0

            lax.fori_loop(0, grp, chunk, 0)
            return 0

        lax.fori_loop(0, ngroups, group, 0)
        plsc.subcore_barrier()
        pltpu.sync_copy(acc.at[pl.ds(r0, rps)], out.at[c, pl.ds(r0, rps)])

    if not scaled:
        def body_nv(table, gidx, sidx, zinit, out, *scr):
            return body(table, gidx, sidx, None, zinit, out, *scr)
        fn = body_nv
    else:
        fn = body

    return pl.kernel(
        fn,
        out_type=jax.ShapeDtypeStruct((NC, out_rows_pad, width), jnp.float32),
        mesh=mesh,
        scratch_types=scratch,
        compiler_params=pltpu.CompilerParams(use_tc_tiling_on_sc=False),
    )


def _seg_pass(table, gather_idx, scatter_idx, out_rows, vals=None, ch=CH, grp=8):
    n_e = gather_idx.shape[0]
    width = table.shape[1]
    n_pad = _round_up(n_e, NW * ch * grp)
    out_rows_pad = _round_up(out_rows + 1, NS * 8)
    junk = out_rows
    pad = n_pad - n_e
    gi = jnp.concatenate(
        [gather_idx.astype(jnp.int32), jnp.zeros((pad,), jnp.int32)]
    ).reshape(n_pad // ch, ch)
    si = jnp.concatenate(
        [scatter_idx.astype(jnp.int32), jnp.full((pad,), junk, jnp.int32)]
    ).reshape(n_pad // ch, ch)
    zinit = jnp.zeros((out_rows_pad, width), jnp.float32)
    k = _make_seg_pass(n_pad, width, out_rows_pad, vals is not None, ch, grp)
    if vals is not None:
        va = jnp.concatenate(
            [vals.astype(jnp.float32), jnp.zeros((pad,), jnp.float32)]
        ).reshape(n_pad // ch, ch)
        parts = k(table, gi, si, va, zinit)
    else:
        parts = k(table, gi, si, zinit)
    return parts[0] + parts[1]


_BM = 1000


def _dot(a, b):
    return jnp.dot(a, b, preferred_element_type=jnp.float32)


def _row_spec(bm, w):
    return pl.BlockSpec((bm, w), lambda i: (i, 0))


def _const_spec(r, c):
    return pl.BlockSpec((r, c), lambda i: (0, 0))


def _tc_psi1(S_v, sum_w, cnt, efeat, w1at, w1bt, b1):
    def body(sv, sw, ct, ef, wa, wb, b, out):
        tmp = _dot(ef[...], wb[...]) + b[...]
        rc = 1.0 / jnp.maximum(ct[...], 1.0)
        out[...] = (_dot(sv[...], wa[...]) + sw[...] * tmp) * rc

    m = S_v.shape[0]
    return pl.pallas_call(
        body,
        grid=(m // _BM,),
        in_specs=[
            _row_spec(_BM, D), _row_spec(_BM, 1), _row_spec(_BM, 1),
            _row_spec(_BM, D), _const_spec(D, D), _const_spec(D, D),
            _const_spec(1, D),
        ],
        out_specs=_row_spec(_BM, D),
        out_shape=jax.ShapeDtypeStruct((m, D), jnp.float32),
    )(S_v, sum_w, cnt, efeat, w1at, w1bt, b1)


def _tc_vout(vf_pre, cnt_src, wvt):
    def body(vp, ct, wv, out):
        rc = 1.0 / jnp.maximum(ct[...], 1.0)
        out[...] = jnp.maximum(_dot(vp[...] * rc, wv[...]), 0.0)

    n = vf_pre.shape[0]
    return pl.pallas_call(
        body,
        grid=(n // _BM,),
        in_specs=[_row_spec(_BM, D), _row_spec(_BM, 1), _const_spec(D, D)],
        out_specs=_row_spec(_BM, D),
        out_shape=jax.ShapeDtypeStruct((n, D), jnp.float32),
    )(vf_pre, cnt_src, wvt)


def _tc_eout(S_vo, ef3_pre, cnt, efeat, w2at, w2bt, b2, wet):
    def body(svo, e3p, ct, ef, wa, wb, b, we, out):
        ctv = ct[...]
        rc = 1.0 / jnp.maximum(ctv, 1.0)
        bb = (_dot(svo[...], wa[...]) + ctv * (_dot(ef[...], wb[...]) + b[...])) * rc
        e3 = e3p[...] * rc + bb
        out[...] = jnp.maximum(_dot(e3, we[...]), 0.0)

    m = S_vo.shape[0]
    return pl.pallas_call(
        body,
        grid=(m // _BM,),
        in_specs=[
            _row_spec(_BM, D), _row_spec(_BM, D), _row_spec(_BM, 1),
            _row_spec(_BM, D), _const_spec(D, D), _const_spec(D, D),
            _const_spec(1, D), _const_spec(D, D),
        ],
        out_specs=_row_spec(_BM, D),
        out_shape=jax.ShapeDtypeStruct((m, D), jnp.float32),
    )(S_vo, ef3_pre, cnt, efeat, w2at, w2bt, b2, wet)


def kernel(vfeat, efeat, invDV, invDE, in_src, in_dst, eMat_row, eMat_col,
           eMat_val, vMat_row, vMat_col, vMat_val, Wv, We, psi1_w, psi1_b,
           psi2_w, psi2_b):
    vfeat = vfeat.astype(jnp.float32)
    efeat = efeat.astype(jnp.float32)
    w1at = psi1_w[:, :D].T
    w1bt = psi1_w[:, D:].T
    w2at = psi2_w[:, :D].T
    w2bt = psi2_w[:, D:].T
    b1 = psi1_b.reshape(1, D)
    b2 = psi2_b.reshape(1, D)

    aug1 = jnp.concatenate(
        [vfeat * invDV[:, None], invDV[:, None], jnp.ones((N, 1), jnp.float32),
         jnp.zeros((N, 14), jnp.float32)], axis=1)
    p1 = _seg_pass(aug1, in_src, in_dst, M, grp=80)
    S_v, sum_w, cnt = p1[:M, :D], p1[:M, D:D + 1], p1[:M, D + 1:D + 2]
    A = _tc_psi1(S_v, sum_w, cnt, efeat, w1at, w1bt, b1)

    As = _seg_pass(A, eMat_col, eMat_row, M, vals=eMat_val, grp=14)
    ef = As[:M, :D] + efeat

    aug3 = jnp.concatenate(
        [efeat * invDE[:, None], jnp.ones((M, 1), jnp.float32),
         jnp.zeros((M, 15), jnp.float32)], axis=1)
    p3 = _seg_pass(aug3, in_dst, in_src, N, grp=80)
    vf2_pre, cnt_src = p3[:N, :D], p3[:N, D:D + 1]
    rcs = 1.0 / jnp.maximum(cnt_src, 1.0)

    vf_pre = _seg_pass(ef, in_dst, in_src, N, grp=80)
    vfo = _tc_vout(vf_pre[:N, :D], cnt_src, Wv.T)

    vf2 = _seg_pass(vf2_pre * rcs, vMat_col, vMat_row, N, vals=vMat_val, grp=26)

    tab45 = jnp.concatenate([vfo, vf2[:N, :D]], axis=1)
    p45 = _seg_pass(tab45, in_src, in_dst, M, ch=64, grp=160)
    S_vo, ef3_pre = p45[:M, :D], p45[:M, D:]
    efo = _tc_eout(S_vo, ef3_pre, cnt, efeat, w2at, w2bt, b2, We.T)

    return (vfo, efo)


# speedup vs baseline: 1.4800x; 1.3594x over previous
import functools

import jax
import jax.numpy as jnp
from jax import lax
from jax.experimental import pallas as pl
from jax.experimental.pallas import tpu as pltpu
from jax.experimental.pallas import tpu_sc as plsc

N = 10000
M = 5000
D = 128

NC = 2
NS = 16
NW = NC * NS
CH = 128


def _round_up(x, m):
    return (x + m - 1) // m * m


@functools.lru_cache(maxsize=None)
def _make_seg_pass(n_edges_pad, width, out_rows_pad, scaled):
    cpw = n_edges_pad // (NW * CH)
    rps = out_rows_pad // NS
    mesh = plsc.VectorSubcoreMesh(
        core_axis_name="c", subcore_axis_name="s", num_cores=NC, num_subcores=NS
    )

    scratch = [
        pltpu.VMEM((CH,), jnp.int32),
        pltpu.VMEM((CH,), jnp.int32),
        pltpu.VMEM((CH, width), jnp.float32),
        pltpu.VMEM_SHARED((out_rows_pad, width), jnp.float32),
        pltpu.SemaphoreType.DMA,
    ]
    if scaled:
        scratch.insert(2, pltpu.VMEM((CH,), jnp.float32))

    def body(table, gidx, sidx, vals, zinit, out, gi_v, si_v, *rest):
        if scaled:
            val_v, rows_v, acc, sem = rest
        else:
            (rows_v, acc, sem) = rest
            val_v = None
        c = lax.axis_index("c")
        s = lax.axis_index("s")
        wid = c * NS + s

        r0 = s * rps
        pltpu.sync_copy(zinit.at[pl.ds(r0, rps)], acc.at[pl.ds(r0, rps)])
        plsc.subcore_barrier()

        def chunk(g, _):
            base = pl.multiple_of((wid * cpw + g) * CH, CH)
            pltpu.sync_copy(gidx.at[pl.ds(base, CH)], gi_v)
            pltpu.sync_copy(sidx.at[pl.ds(base, CH)], si_v)
            pltpu.async_copy(table.at[gi_v], rows_v, sem).wait()
            if scaled:
                pltpu.sync_copy(vals.at[pl.ds(base, CH)], val_v)

                def scale_blk(b, _):
                    base16 = b * 16
                    vv = val_v[pl.ds(base16, 16)]
                    for i in range(16):
                        v = vv[i]
                        for j in range(width // 16):
                            sl = pl.ds(j * 16, 16)
                            rows_v[base16 + i, sl] = rows_v[base16 + i, sl] * v
                    return 0

                lax.fori_loop(0, CH // 16, scale_blk, 0)
            pltpu.sync_copy(rows_v, acc.at[si_v], add=True)
            return 0

        lax.fori_loop(0, cpw, chunk, 0)
        plsc.subcore_barrier()
        pltpu.sync_copy(acc.at[pl.ds(r0, rps)], out.at[c, pl.ds(r0, rps)])

    if not scaled:
        def body_nv(table, gidx, sidx, zinit, out, *scr):
            return body(table, gidx, sidx, None, zinit, out, *scr)
        fn = body_nv
    else:
        fn = body

    return pl.kernel(
        fn,
        out_type=jax.ShapeDtypeStruct((NC, out_rows_pad, width), jnp.float32),
        mesh=mesh,
        scratch_types=scratch,
        compiler_params=pltpu.CompilerParams(use_tc_tiling_on_sc=False),
    )


def _seg_pass(table, gather_idx, scatter_idx, out_rows, vals=None):
    n_e = gather_idx.shape[0]
    width = table.shape[1]
    n_pad = _round_up(n_e, NW * CH)
    out_rows_pad = _round_up(out_rows + 1, NS * 8)
    junk = out_rows
    pad = n_pad - n_e
    gi = jnp.concatenate([gather_idx.astype(jnp.int32), jnp.zeros((pad,), jnp.int32)])
    si = jnp.concatenate(
        [scatter_idx.astype(jnp.int32), jnp.full((pad,), junk, jnp.int32)]
    )
    zinit = jnp.zeros((out_rows_pad, width), jnp.float32)
    k = _make_seg_pass(n_pad, width, out_rows_pad, vals is not None)
    if vals is not None:
        va = jnp.concatenate([vals.astype(jnp.float32), jnp.zeros((pad,), jnp.float32)])
        parts = k(table, gi, si, va, zinit)
    else:
        parts = k(table, gi, si, zinit)
    return parts[0] + parts[1]


_BM = 1000


def _dot(a, b):
    return jnp.dot(a, b, preferred_element_type=jnp.float32)


def _row_spec(bm, w):
    return pl.BlockSpec((bm, w), lambda i: (i, 0))


def _const_spec(r, c):
    return pl.BlockSpec((r, c), lambda i: (0, 0))


def _tc_psi1(S_v, sum_w, cnt, efeat, w1at, w1bt, b1):
    def body(sv, sw, ct, ef, wa, wb, b, out):
        tmp = _dot(ef[...], wb[...]) + b[...]
        rc = 1.0 / jnp.maximum(ct[...], 1.0)
        out[...] = (_dot(sv[...], wa[...]) + sw[...] * tmp) * rc

    m = S_v.shape[0]
    return pl.pallas_call(
        body,
        grid=(m // _BM,),
        in_specs=[
            _row_spec(_BM, D), _row_spec(_BM, 1), _row_spec(_BM, 1),
            _row_spec(_BM, D), _const_spec(D, D), _const_spec(D, D),
            _const_spec(1, D),
        ],
        out_specs=_row_spec(_BM, D),
        out_shape=jax.ShapeDtypeStruct((m, D), jnp.float32),
    )(S_v, sum_w, cnt, efeat, w1at, w1bt, b1)


def _tc_vout(vf_pre, cnt_src, wvt):
    def body(vp, ct, wv, out):
        rc = 1.0 / jnp.maximum(ct[...], 1.0)
        out[...] = jnp.maximum(_dot(vp[...] * rc, wv[...]), 0.0)

    n = vf_pre.shape[0]
    return pl.pallas_call(
        body,
        grid=(n // _BM,),
        in_specs=[_row_spec(_BM, D), _row_spec(_BM, 1), _const_spec(D, D)],
        out_specs=_row_spec(_BM, D),
        out_shape=jax.ShapeDtypeStruct((n, D), jnp.float32),
    )(vf_pre, cnt_src, wvt)


def _tc_eout(S_vo, ef3_pre, cnt, efeat, w2at, w2bt, b2, wet):
    def body(svo, e3p, ct, ef, wa, wb, b, we, out):
        ctv = ct[...]
        rc = 1.0 / jnp.maximum(ctv, 1.0)
        bb = (_dot(svo[...], wa[...]) + ctv * (_dot(ef[...], wb[...]) + b[...])) * rc
        e3 = e3p[...] * rc + bb
        out[...] = jnp.maximum(_dot(e3, we[...]), 0.0)

    m = S_vo.shape[0]
    return pl.pallas_call(
        body,
        grid=(m // _BM,),
        in_specs=[
            _row_spec(_BM, D), _row_spec(_BM, D), _row_spec(_BM, 1),
            _row_spec(_BM, D), _const_spec(D, D), _const_spec(D, D),
            _const_spec(1, D), _const_spec(D, D),
        ],
        out_specs=_row_spec(_BM, D),
        out_shape=jax.ShapeDtypeStruct((m, D), jnp.float32),
    )(S_vo, ef3_pre, cnt, efeat, w2at, w2bt, b2, wet)


def kernel(vfeat, efeat, invDV, invDE, in_src, in_dst, eMat_row, eMat_col,
           eMat_val, vMat_row, vMat_col, vMat_val, Wv, We, psi1_w, psi1_b,
           psi2_w, psi2_b):
    vfeat = vfeat.astype(jnp.float32)
    efeat = efeat.astype(jnp.float32)
    w1at = psi1_w[:, :D].T
    w1bt = psi1_w[:, D:].T
    w2at = psi2_w[:, :D].T
    w2bt = psi2_w[:, D:].T
    b1 = psi1_b.reshape(1, D)
    b2 = psi2_b.reshape(1, D)

    aug1 = jnp.concatenate(
        [vfeat * invDV[:, None], invDV[:, None], jnp.ones((N, 1), jnp.float32),
         jnp.zeros((N, 14), jnp.float32)], axis=1)
    p1 = _seg_pass(aug1, in_src, in_dst, M)
    S_v, sum_w, cnt = p1[:M, :D], p1[:M, D:D + 1], p1[:M, D + 1:D + 2]
    A = _tc_psi1(S_v, sum_w, cnt, efeat, w1at, w1bt, b1)

    As = _seg_pass(A, eMat_col, eMat_row, M, vals=eMat_val)
    ef = As[:M, :D] + efeat

    aug3 = jnp.concatenate(
        [efeat * invDE[:, None], jnp.ones((M, 1), jnp.float32),
         jnp.zeros((M, 15), jnp.float32)], axis=1)
    p3 = _seg_pass(aug3, in_dst, in_src, N)
    vf2_pre, cnt_src = p3[:N, :D], p3[:N, D:D + 1]
    rcs = 1.0 / jnp.maximum(cnt_src, 1.0)

    vf_pre = _seg_pass(ef, in_dst, in_src, N)
    vfo = _tc_vout(vf_pre[:N, :D], cnt_src, Wv.T)

    vf2 = _seg_pass(vf2_pre * rcs, vMat_col, vMat_row, N, vals=vMat_val)

    tab45 = jnp.concatenate([vfo, vf2[:N, :D]], axis=1)
    p45 = _seg_pass(tab45, in_src, in_dst, M)
    S_vo, ef3_pre = p45[:M, :D], p45[:M, D:]
    efo = _tc_eout(S_vo, ef3_pre, cnt, efeat, w2at, w2bt, b2, We.T)

    return (vfo, efo)
